# Initial kernel scaffold; baseline (speedup 1.0000x reference)
import jax
import jax.numpy as jnp
from jax.experimental import pallas as pl


def kernel(emb, lin_w, lin_b, ct1_w, ct2_w, ct3_w, ct4_w, c5_w, c6_w, bn1_g, bn1_b, bn2_g, bn2_b, bn3_g, bn3_b, bn4_g, bn4_b, bn5_g, bn5_b, z, labels):
    raise NotImplementedError("write your pallas kernel here")


# speedup vs baseline: 3.6770x; 3.6770x over previous
import functools

import jax
import jax.numpy as jnp
from jax.experimental import pallas as pl
from jax.experimental.pallas import tpu as pltpu

_PARITIES = ((0, 0), (0, 1), (1, 0), (1, 1))
_TAP = {(0, 0): 3, (0, 1): 1, (1, 1): 2, (1, 2): 0}


def _bn_scale_shift(col_sum, col_sumsq, count, gamma, beta, eps=1e-5):
    mean = col_sum / count
    var = jnp.maximum(col_sumsq / count - mean * mean, 0.0)
    scale = gamma * jax.lax.rsqrt(var + eps)
    shift = beta - mean * scale
    return scale, shift


def _parity_weights(w_t):
    mats = []
    for rh, rw in _PARITIES:
        dhs = (0, 1) if rh == 0 else (1, 2)
        dws = (0, 1) if rw == 0 else (1, 2)
        blocks = [w_t[:, :, _TAP[(rh, dh)], _TAP[(rw, dw)]]
                  for dh in dhs for dw in dws]
        mats.append(jnp.concatenate(blocks, axis=0))
    return jnp.stack(mats).astype(jnp.bfloat16)


def _dense_deconv_weights(w_t):
    Cin, Cout = w_t.shape[0], w_t.shape[1]
    W = jnp.zeros((3, 3, Cin, 2, 2, Cout), w_t.dtype)
    for (rh, dh), kh in _TAP.items():
        for (rw, dw), kw in _TAP.items():
            W = W.at[dh, dw, :, rh, rw, :].set(w_t[:, :, kh, kw])
    return W.reshape(9 * Cin, 4 * Cout).astype(jnp.bfloat16)


def _conv_weights(w):
    return jnp.transpose(w, (2, 3, 1, 0)).reshape(-1, w.shape[0]).astype(
        jnp.bfloat16)


def _shifted_planes(x, H, W, ipb):
    C = x.shape[-1]
    xw = [x[:, :, dw:dw + W, :] for dw in range(3)]
    return {(dh, dw): xw[dw][:, dh:dh + H].reshape(ipb * H * W, C)
            for dh in range(3) for dw in range(3)}


def _emit(o_ref, s_ref, q_ref, y):
    yb = y.astype(jnp.bfloat16)
    o_ref[...] = yb
    yf = yb.astype(jnp.float32)
    s_ref[...] = jnp.sum(yf, axis=0).reshape(s_ref.shape)
    q_ref[...] = jnp.sum(yf * yf, axis=0).reshape(q_ref.shape)


def _deconv_body(x_ref, w_ref, o_ref, s_ref, q_ref, *, H, W, ipb):
    planes = _shifted_planes(x_ref[...], H, W, ipb)
    outs = []
    for p, (rh, rw) in enumerate(_PARITIES):
        dhs = (0, 1) if rh == 0 else (1, 2)
        dws = (0, 1) if rw == 0 else (1, 2)
        col = jnp.concatenate([planes[(a, b)] for a in dhs for b in dws],
                              axis=-1)
        outs.append(jnp.dot(col, w_ref[p],
                            preferred_element_type=jnp.float32))
    _emit(o_ref, s_ref, q_ref, jnp.concatenate(outs, axis=-1))


def _conv9_body(x_ref, w_ref, o_ref, s_ref, q_ref, *, H, W, ipb):
    planes = _shifted_planes(x_ref[...], H, W, ipb)
    col = jnp.concatenate([planes[(dh, dw)]
                           for dh in range(3) for dw in range(3)], axis=-1)
    y = jnp.dot(col, w_ref[...], preferred_element_type=jnp.float32)
    _emit(o_ref, s_ref, q_ref, y)


def _conv9_tanh_body(x_ref, w_ref, o_ref, *, H, W, ipb):
    planes = _shifted_planes(x_ref[...], H, W, ipb)
    col = jnp.concatenate([planes[(dh, dw)]
                           for dh in range(3) for dw in range(3)], axis=-1)
    y = jnp.dot(col, w_ref[...], preferred_element_type=jnp.float32)
    o_ref[...] = jnp.tanh(y)


def _pick_ipb(N, HW):
    ipb = max(1, min(N // 2, 2048 // HW))
    while N % ipb:
        ipb -= 1
    return ipb


def _launch(body, xpad, wmat, out_cols, out_dtype, with_stats):
    N, Hp, Wp, C = xpad.shape
    H, W = Hp - 2, Wp - 2
    ipb = _pick_ipb(N, H * W)
    G = N // ipb
    kern = functools.partial(body, H=H, W=W, ipb=ipb)
    in_specs = [
        pl.BlockSpec((ipb, Hp, Wp, C), lambda i: (i, 0, 0, 0)),
        pl.BlockSpec(wmat.shape, lambda i: (0,) * wmat.ndim),
    ]
    out_shapes = [jax.ShapeDtypeStruct((N * H * W, out_cols), out_dtype)]
    out_specs = [pl.BlockSpec((ipb * H * W, out_cols), lambda i: (i, 0))]
    if with_stats:
        out_shapes += [jax.ShapeDtypeStruct((G, 1, out_cols),
                                            jnp.float32)] * 2
        out_specs += [pl.BlockSpec((1, 1, out_cols), lambda i: (i, 0, 0))] * 2
    out = pl.pallas_call(
        kern,
        out_shape=tuple(out_shapes),
        grid=(G,),
        in_specs=in_specs,
        out_specs=tuple(out_specs),
        compiler_params=pltpu.CompilerParams(
            dimension_semantics=("parallel",)),
    )(xpad, wmat)
    return out if with_stats else out[0]


def _bn_d2s_pad(out2d, s, q, gamma, beta, N, H, W):
    Cout = gamma.shape[0]
    s = s.sum(axis=(0, 1)).reshape(4, Cout).sum(axis=0)
    q = q.sum(axis=(0, 1)).reshape(4, Cout).sum(axis=0)
    scale, shift = _bn_scale_shift(s, q, 4.0 * N * H * W, gamma, beta)
    y = out2d.reshape(N, H, W, 2, 2, Cout)
    y = y.transpose(0, 1, 3, 2, 4, 5).reshape(N, 2 * H, 2 * W, Cout)
    y = jnp.maximum(y.astype(jnp.float32) * scale + shift, 0.0)
    return jnp.pad(y.astype(jnp.bfloat16), ((0, 0), (1, 1), (1, 1), (0, 0)))


def _bn_relu_pad(out2d, s, q, gamma, beta, N, H, W):
    Cout = gamma.shape[0]
    scale, shift = _bn_scale_shift(s.sum(axis=(0, 1)), q.sum(axis=(0, 1)),
                                   float(N * H * W), gamma, beta)
    y = out2d.reshape(N, H, W, Cout).astype(jnp.float32)
    y = jnp.maximum(y * scale + shift, 0.0)
    return jnp.pad(y.astype(jnp.bfloat16), ((0, 0), (1, 1), (1, 1), (0, 0)))


def _linear_body(a_ref, w_ref, b_ref, o_ref):
    o_ref[...] = (jnp.dot(a_ref[...], w_ref[...],
                          preferred_element_type=jnp.float32)
                  + b_ref[...]).astype(jnp.bfloat16)


def _linear(a, w, b):
    Bm, K = a.shape
    Nn = w.shape[1]
    nt = 2
    return pl.pallas_call(
        _linear_body,
        out_shape=jax.ShapeDtypeStruct((Bm, Nn), jnp.bfloat16),
        grid=(nt,),
        in_specs=[pl.BlockSpec((Bm, K), lambda i: (0, 0)),
                  pl.BlockSpec((K, Nn // nt), lambda i: (0, i)),
                  pl.BlockSpec((1, Nn // nt), lambda i: (0, i))],
        out_specs=pl.BlockSpec((Bm, Nn // nt), lambda i: (0, i)),
        compiler_params=pltpu.CompilerParams(
            dimension_semantics=("parallel",)),
    )(a, w, b)


def kernel(emb, lin_w, lin_b, ct1_w, ct2_w, ct3_w, ct4_w, c5_w, c6_w,
           bn1_g, bn1_b, bn2_g, bn2_b, bn3_g, bn3_b, bn4_g, bn4_b,
           bn5_g, bn5_b, z, labels):
    B = z.shape[0]
    ngf = c5_w.shape[0]
    C0 = 8 * ngf
    nc = c6_w.shape[0]

    x = jnp.concatenate([emb[labels], z.reshape(B, -1)],
                        axis=1).astype(jnp.bfloat16)
    wl = lin_w.reshape(C0, 16, -1).transpose(1, 0, 2).reshape(16 * C0, -1).T
    bl = lin_b.reshape(C0, 16).T.reshape(1, 16 * C0)
    h = _linear(x, wl.astype(jnp.bfloat16), bl.astype(jnp.float32))
    xp = jnp.pad(h.reshape(B, 4, 4, C0), ((0, 0), (1, 1), (1, 1), (0, 0)))

    for w_t, g, b in ((ct1_w, bn1_g, bn1_b), (ct2_w, bn2_g, bn2_b),
                      (ct3_w, bn3_g, bn3_b)):
        N, Hp, Wp, _ = xp.shape
        H, W = Hp - 2, Wp - 2
        Cout = w_t.shape[1]
        out2d, s, q = _launch(_deconv_body, xp, _parity_weights(w_t),
                              4 * Cout, jnp.bfloat16, True)
        xp = _bn_d2s_pad(out2d, s, q, g, b, N, H, W)

    N, Hp, Wp, _ = xp.shape
    H, W = Hp - 2, Wp - 2
    Cout = ct4_w.shape[1]
    out2d, s, q = _launch(_conv9_body, xp, _dense_deconv_weights(ct4_w),
                          4 * Cout, jnp.bfloat16, True)
    xp = _bn_d2s_pad(out2d, s, q, bn4_g, bn4_b, N, H, W)

    N, Hp, Wp, _ = xp.shape
    H, W = Hp - 2, Wp - 2
    out2d, s, q = _launch(_conv9_body, xp, _conv_weights(c5_w),
                          ngf, jnp.bfloat16, True)
    xp = _bn_relu_pad(out2d, s, q, bn5_g, bn5_b, N, H, W)

    out2d = _launch(_conv9_tanh_body, xp, _conv_weights(c6_w),
                    nc, jnp.float32, False)
    img = out2d.reshape(N, H, W, nc)
    return img.transpose(0, 3, 1, 2)


# speedup vs baseline: 6.6692x; 1.8138x over previous
import functools

import jax
import jax.numpy as jnp
from jax.experimental import pallas as pl
from jax.experimental.pallas import tpu as pltpu

_PARITIES = ((0, 0), (0, 1), (1, 0), (1, 1))
_TAP = {(0, 0): 3, (0, 1): 1, (1, 1): 2, (1, 2): 0}


def _bn_scale_shift(col_sum, col_sumsq, count, gamma, beta, eps=1e-5):
    mean = col_sum / count
    var = jnp.maximum(col_sumsq / count - mean * mean, 0.0)
    scale = gamma * jax.lax.rsqrt(var + eps)
    shift = beta - mean * scale
    return scale, shift


def _parity_weights(w_t):
    mats = []
    for rh, rw in _PARITIES:
        dhs = (0, 1) if rh == 0 else (1, 2)
        dws = (0, 1) if rw == 0 else (1, 2)
        blocks = [w_t[:, :, _TAP[(rh, dh)], _TAP[(rw, dw)]]
                  for dh in dhs for dw in dws]
        mats.append(jnp.concatenate(blocks, axis=0))
    return jnp.stack(mats).astype(jnp.bfloat16)


def _dense_deconv_weights(w_t):
    Cin, Cout = w_t.shape[0], w_t.shape[1]
    zero = jnp.zeros((Cin, Cout), w_t.dtype)
    rows = []
    for dh in range(3):
        for dw in range(3):
            cols = []
            for rh, rw in _PARITIES:
                ok_h = (rh, dh) in _TAP
                ok_w = (rw, dw) in _TAP
                cols.append(w_t[:, :, _TAP[(rh, dh)], _TAP[(rw, dw)]]
                            if (ok_h and ok_w) else zero)
            rows.append(jnp.concatenate(cols, axis=1))
    return jnp.concatenate(rows, axis=0).astype(jnp.bfloat16)


def _conv_weights(w):
    return jnp.transpose(w, (2, 3, 1, 0)).reshape(-1, w.shape[0]).astype(
        jnp.bfloat16)


def _affine_relu_pad(x, sc_ref, sh_ref):
    y = jnp.maximum(x.astype(jnp.float32) * sc_ref[...] + sh_ref[...], 0.0)
    return jnp.pad(y.astype(jnp.bfloat16),
                   ((0, 0), (1, 1), (1, 1), (0, 0)))


def _shifted_planes(xp, H, W, ipb):
    C = xp.shape[-1]
    xw = [xp[:, :, dw:dw + W, :] for dw in range(3)]
    return {(dh, dw): xw[dw][:, dh:dh + H].reshape(ipb * H * W, C)
            for dh in range(3) for dw in range(3)}


def _stats(s_ref, q_ref, ypacked):
    yf = ypacked.astype(jnp.float32)
    s_ref[...] = jnp.sum(yf, axis=0).reshape(s_ref.shape)
    q_ref[...] = jnp.sum(yf * yf, axis=0).reshape(q_ref.shape)


def _w_interleave(a, b):
    ar = jnp.repeat(a, 2, axis=2)
    br = jnp.repeat(b, 2, axis=2)
    s = jax.lax.broadcasted_iota(jnp.int32, ar.shape, 2)
    return jnp.where(s % 2 == 0, ar, br)


def _store_d2s(o_ref, ys, H, W, ipb):
    C = ys[0].shape[-1]
    y4 = [y.reshape(ipb, H, W, C) for y in ys]
    o_ref[:, :, 0] = _w_interleave(y4[0], y4[1])
    o_ref[:, :, 1] = _w_interleave(y4[2], y4[3])


def _parity_matmuls(planes, w_ref):
    ys = []
    for p, (rh, rw) in enumerate(_PARITIES):
        dhs = (0, 1) if rh == 0 else (1, 2)
        dws = (0, 1) if rw == 0 else (1, 2)
        col = jnp.concatenate([planes[(a, b)] for a in dhs for b in dws],
                              axis=-1)
        ys.append(jnp.dot(col, w_ref[p],
                          preferred_element_type=jnp.float32)
                  .astype(jnp.bfloat16))
    return ys


def _col9_matmul(planes, w_ref):
    col = jnp.concatenate([planes[(dh, dw)]
                           for dh in range(3) for dw in range(3)], axis=-1)
    return jnp.dot(col, w_ref[...], preferred_element_type=jnp.float32)


def _lin_deconv_body(a_ref, wl_ref, bl_ref, w_ref, o_ref, s_ref, q_ref, *,
                     H, W, ipb):
    h = (jnp.dot(a_ref[...], wl_ref[...], preferred_element_type=jnp.float32)
         + bl_ref[...]).astype(jnp.bfloat16)
    x = h.reshape(ipb, H, W, h.shape[-1] // (H * W))
    xp = jnp.pad(x, ((0, 0), (1, 1), (1, 1), (0, 0)))
    planes = _shifted_planes(xp, H, W, ipb)
    ys = _parity_matmuls(planes, w_ref)
    _stats(s_ref, q_ref, jnp.concatenate(ys, axis=-1))
    _store_d2s(o_ref, ys, H, W, ipb)


def _deconv_body(x_ref, sc_ref, sh_ref, w_ref, o_ref, s_ref, q_ref, *,
                 H, W, ipb):
    xp = _affine_relu_pad(x_ref[...], sc_ref, sh_ref)
    planes = _shifted_planes(xp, H, W, ipb)
    ys = _parity_matmuls(planes, w_ref)
    _stats(s_ref, q_ref, jnp.concatenate(ys, axis=-1))
    _store_d2s(o_ref, ys, H, W, ipb)


def _deconv9_body(x_ref, sc_ref, sh_ref, w_ref, o_ref, s_ref, q_ref, *,
                  H, W, ipb):
    xp = _affine_relu_pad(x_ref[...], sc_ref, sh_ref)
    planes = _shifted_planes(xp, H, W, ipb)
    y = _col9_matmul(planes, w_ref).astype(jnp.bfloat16)
    _stats(s_ref, q_ref, y)
    C = y.shape[-1] // 4
    ys = [y[:, p * C:(p + 1) * C] for p in range(4)]
    _store_d2s(o_ref, ys, H, W, ipb)


def _conv_body(x_ref, sc_ref, sh_ref, w_ref, o_ref, s_ref, q_ref, *,
               H, W, ipb):
    xp = _affine_relu_pad(x_ref[...], sc_ref, sh_ref)
    planes = _shifted_planes(xp, H, W, ipb)
    y = _col9_matmul(planes, w_ref).astype(jnp.bfloat16)
    _stats(s_ref, q_ref, y)
    o_ref[...] = y.reshape(ipb, H, W, y.shape[-1])


def _conv_tanh_body(x_ref, sc_ref, sh_ref, w_ref, o_ref, *, H, W, ipb):
    xp = _affine_relu_pad(x_ref[...], sc_ref, sh_ref)
    planes = _shifted_planes(xp, H, W, ipb)
    o_ref[...] = jnp.tanh(_col9_matmul(planes, w_ref))


def _pick_ipb(N, HW):
    ipb = max(1, min(N // 2, 1024 // HW))
    while N % ipb:
        ipb -= 1
    return ipb


def _launch_conv(body, x, scale, shift, wmat, Cout, out_kind):
    N, H, W, C = x.shape
    ipb = _pick_ipb(N, H * W)
    G = N // ipb
    kern = functools.partial(body, H=H, W=W, ipb=ipb)
    in_specs = [
        pl.BlockSpec((ipb, H, W, C), lambda i: (i, 0, 0, 0)),
        pl.BlockSpec((1, C), lambda i: (0, 0)),
        pl.BlockSpec((1, C), lambda i: (0, 0)),
        pl.BlockSpec(wmat.shape, lambda i: (0,) * wmat.ndim),
    ]
    if out_kind == "d2s":
        out_shapes = [jax.ShapeDtypeStruct((N, H, 2, 2 * W, Cout),
                                           jnp.bfloat16)]
        out_specs = [pl.BlockSpec((ipb, H, 2, 2 * W, Cout),
                                  lambda i: (i, 0, 0, 0, 0))]
        stat_c = 4 * Cout
    elif out_kind == "plane":
        out_shapes = [jax.ShapeDtypeStruct((N, H, W, Cout), jnp.bfloat16)]
        out_specs = [pl.BlockSpec((ipb, H, W, Cout),
                                  lambda i: (i, 0, 0, 0))]
        stat_c = Cout
    else:
        out_shapes = [jax.ShapeDtypeStruct((N * H * W, Cout), jnp.float32)]
        out_specs = [pl.BlockSpec((ipb * H * W, Cout), lambda i: (i, 0))]
        stat_c = 0
    if stat_c:
        out_shapes += [jax.ShapeDtypeStruct((G, 1, stat_c), jnp.float32)] * 2
        out_specs += [pl.BlockSpec((1, 1, stat_c), lambda i: (i, 0, 0))] * 2
    out = pl.pallas_call(
        kern,
        out_shape=tuple(out_shapes),
        grid=(G,),
        in_specs=in_specs,
        out_specs=tuple(out_specs),
        compiler_params=pltpu.CompilerParams(
            dimension_semantics=("parallel",)),
    )(x, scale, shift, wmat)
    if out_kind == "d2s":
        return out[0].reshape(N, 2 * H, 2 * W, Cout), out[1], out[2]
    return out if stat_c else out[0]


def kernel(emb, lin_w, lin_b, ct1_w, ct2_w, ct3_w, ct4_w, c5_w, c6_w,
           bn1_g, bn1_b, bn2_g, bn2_b, bn3_g, bn3_b, bn4_g, bn4_b,
           bn5_g, bn5_b, z, labels):
    B = z.shape[0]
    ngf = c5_w.shape[0]
    C0 = 8 * ngf
    nc = c6_w.shape[0]

    a = jnp.concatenate([emb[labels], z.reshape(B, -1)],
                        axis=1).astype(jnp.bfloat16)
    wl = lin_w.reshape(C0, 16, -1).transpose(1, 0, 2).reshape(16 * C0, -1).T
    bl = lin_b.reshape(C0, 16).T.reshape(1, 16 * C0)

    H, W = 4, 4
    Cout = ct1_w.shape[1]
    ipb = _pick_ipb(B, H * W)
    G = B // ipb
    out1 = pl.pallas_call(
        functools.partial(_lin_deconv_body, H=H, W=W, ipb=ipb),
        out_shape=(
            jax.ShapeDtypeStruct((B, H, 2, 2 * W, Cout), jnp.bfloat16),
            jax.ShapeDtypeStruct((G, 1, 4 * Cout), jnp.float32),
            jax.ShapeDtypeStruct((G, 1, 4 * Cout), jnp.float32),
        ),
        grid=(G,),
        in_specs=[
            pl.BlockSpec((ipb, a.shape[1]), lambda i: (i, 0)),
            pl.BlockSpec(wl.shape, lambda i: (0, 0)),
            pl.BlockSpec((1, 16 * C0), lambda i: (0, 0)),
            pl.BlockSpec((4, 4 * C0, Cout), lambda i: (0, 0, 0)),
        ],
        out_specs=(
            pl.BlockSpec((ipb, H, 2, 2 * W, Cout),
                         lambda i: (i, 0, 0, 0, 0)),
            pl.BlockSpec((1, 1, 4 * Cout), lambda i: (i, 0, 0)),
            pl.BlockSpec((1, 1, 4 * Cout), lambda i: (i, 0, 0)),
        ),
        compiler_params=pltpu.CompilerParams(
            dimension_semantics=("parallel",)),
    )(a, wl.astype(jnp.bfloat16), bl.astype(jnp.float32),
      _parity_weights(ct1_w))
    x = out1[0].reshape(B, 2 * H, 2 * W, Cout)
    s, q = out1[1], out1[2]

    def coeffs(s, q, count, gamma, beta, fold4):
        s = s.sum(axis=(0, 1))
        q = q.sum(axis=(0, 1))
        if fold4:
            Cc = gamma.shape[0]
            s = s.reshape(4, Cc).sum(axis=0)
            q = q.reshape(4, Cc).sum(axis=0)
        sc, sh = _bn_scale_shift(s, q, count, gamma, beta)
        return sc.reshape(1, -1), sh.reshape(1, -1)

    for w_t, (g, b) in ((ct2_w, (bn1_g, bn1_b)), (ct3_w, (bn2_g, bn2_b))):
        N, H2, W2, C = x.shape
        sc, sh = coeffs(s, q, float(N * H2 * W2), g, b, True)
        Cout = w_t.shape[1]
        x, s, q = _launch_conv(
            _deconv_body, x, sc, sh, _parity_weights(w_t), Cout, "d2s")

    N, H2, W2, C = x.shape
    sc, sh = coeffs(s, q, float(N * H2 * W2), bn3_g, bn3_b, True)
    Cout = ct4_w.shape[1]
    x, s, q = _launch_conv(
        _deconv9_body, x, sc, sh, _dense_deconv_weights(ct4_w), Cout, "d2s")

    N, H2, W2, C = x.shape
    sc, sh = coeffs(s, q, float(N * H2 * W2), bn4_g, bn4_b, True)
    x, s, q = _launch_conv(
        _conv_body, x, sc, sh, _conv_weights(c5_w), ngf, "plane")

    N, H2, W2, C = x.shape
    sc, sh = coeffs(s, q, float(N * H2 * W2), bn5_g, bn5_b, False)
    out = _launch_conv(
        _conv_tanh_body, x, sc, sh, _conv_weights(c6_w), nc, "rows")
    img = out.reshape(N, H2, W2, nc)
    return img.transpose(0, 3, 1, 2)


# speedup vs baseline: 8.2731x; 1.2405x over previous
import functools

import jax
import jax.numpy as jnp
from jax.experimental import pallas as pl
from jax.experimental.pallas import tpu as pltpu

_PARITIES = ((0, 0), (0, 1), (1, 0), (1, 1))
_TAP = {(0, 0): 3, (0, 1): 1, (1, 1): 2, (1, 2): 0}


def _bn_scale_shift(col_sum, col_sumsq, count, gamma, beta, eps=1e-5):
    mean = col_sum / count
    var = jnp.maximum(col_sumsq / count - mean * mean, 0.0)
    scale = gamma * jax.lax.rsqrt(var + eps)
    shift = beta - mean * scale
    return scale, shift


def _parity_weights(w_t):
    mats = []
    for rh, rw in _PARITIES:
        dhs = (0, 1) if rh == 0 else (1, 2)
        dws = (0, 1) if rw == 0 else (1, 2)
        taps = [jnp.concatenate(
            [w_t[:, :, _TAP[(rh, dh)], _TAP[(rw, dw)]] for dw in dws],
            axis=0) for dh in dhs]
        mats.append(jnp.stack(taps))
    return jnp.stack(mats).astype(jnp.bfloat16)


def _dense_deconv_weights(w_t):
    Cin, Cout = w_t.shape[0], w_t.shape[1]
    zero = jnp.zeros((Cin, Cout), w_t.dtype)
    blocks = []
    for dh in range(3):
        rows = []
        for dw in range(3):
            cols = [w_t[:, :, _TAP[(rh, dh)], _TAP[(rw, dw)]]
                    if ((rh, dh) in _TAP and (rw, dw) in _TAP) else zero
                    for rh, rw in _PARITIES]
            rows.append(jnp.concatenate(cols, axis=1))
        blocks.append(jnp.concatenate(rows, axis=0))
    return jnp.stack(blocks).astype(jnp.bfloat16)


def _packed_conv_weights(w):
    Cout, Cin = w.shape[0], w.shape[1]
    zero = jnp.zeros((Cin, Cout), w.dtype)
    blocks = []
    for dh in range(3):
        rows = []
        for st in range(4):
            cols = [jnp.transpose(w[:, :, dh, st - sp])
                    if 0 <= st - sp <= 2 else zero for sp in range(2)]
            rows.append(jnp.concatenate(cols, axis=1))
        blocks.append(jnp.concatenate(rows, axis=0))
    return jnp.stack(blocks).astype(jnp.bfloat16)


def _affine_relu(x, sc_ref, sh_ref):
    y = jnp.maximum(x.astype(jnp.float32) * sc_ref[...] + sh_ref[...], 0.0)
    return y.astype(jnp.bfloat16)


def _conv_cols(x):
    z = jnp.zeros_like(x[:, :1])
    xh = jnp.concatenate([z, x, z], axis=1)
    zc = jnp.zeros_like(xh[:, :, :1, :])
    xl = jnp.concatenate([zc, xh[:, :, :-1, :]], axis=2)
    xr = jnp.concatenate([xh[:, :, 1:, :], zc], axis=2)
    return xl, xh, xr


def _parity_matmuls(x, w_ref, H, W, ipb):
    xl, xh, xr = _conv_cols(x)
    cw01 = jnp.concatenate([xl, xh], axis=-1)
    cw12 = jnp.concatenate([xh, xr], axis=-1)
    cws = (cw01, cw12)
    ys = []
    for p, (rh, rw) in enumerate(_PARITIES):
        dhs = (0, 1) if rh == 0 else (1, 2)
        cw = cws[rw]
        acc = None
        for t, dh in enumerate(dhs):
            m = cw[:, dh:dh + H].reshape(ipb * H * W, cw.shape[-1])
            d = jnp.dot(m, w_ref[p, t], preferred_element_type=jnp.float32)
            acc = d if acc is None else acc + d
        ys.append(acc.astype(jnp.bfloat16))
    return ys


def _col9_matmul(x, w_ref, H, W, ipb):
    xl, xh, xr = _conv_cols(x)
    cw = jnp.concatenate([xl, xh, xr], axis=-1)
    acc = None
    for dh in range(3):
        m = cw[:, dh:dh + H].reshape(ipb * H * W, cw.shape[-1])
        d = jnp.dot(m, w_ref[dh], preferred_element_type=jnp.float32)
        acc = d if acc is None else acc + d
    return acc


def _stats4(s_ref, q_ref, ys):
    ss, qs = [], []
    for y in ys:
        yf = y.astype(jnp.float32)
        ss.append(jnp.sum(yf, axis=0, keepdims=True))
        qs.append(jnp.sum(yf * yf, axis=0, keepdims=True))
    s_ref[...] = jnp.concatenate(ss, axis=-1).reshape(s_ref.shape)
    q_ref[...] = jnp.concatenate(qs, axis=-1).reshape(q_ref.shape)


def _store_d2s(o_ref, ys, H, W, ipb):
    C = ys[0].shape[-1]
    for p, (rh, rw) in enumerate(_PARITIES):
        y = ys[p].astype(jnp.float32).reshape(ipb, H, W, C // 128, 128)
        o_ref[:, :, rh, pl.ds(rw, W, 2)] = y


def _store_d2s_packed(o_ref, ys, H, W, ipb):
    C = ys[0].shape[-1]
    for rh in range(2):
        y = jnp.concatenate([ys[2 * rh], ys[2 * rh + 1]], axis=-1)
        o_ref[:, :, rh] = y.reshape(ipb, H, W, 2 * C)


def _packed_cols(x, R, W, ipb):
    C2 = x.shape[-1]
    C = C2 // 2
    z = jnp.zeros_like(x[:, :1])
    xh = jnp.concatenate([z, x, z], axis=1)
    zc = jnp.zeros_like(xh[:, :, :1, :])
    xl = jnp.concatenate([zc, xh[:, :, :-1, :]], axis=2)
    xr = jnp.concatenate([xh[:, :, 1:, :], zc], axis=2)
    return jnp.concatenate(
        [xl[..., C:], xh[..., :C], xh[..., C:], xr[..., :C]], axis=-1)


def _lin_deconv_body(a_ref, wl_ref, bl_ref, w_ref, o_ref, s_ref, q_ref, *,
                     H, W, ipb):
    h = (jnp.dot(a_ref[...], wl_ref[...], preferred_element_type=jnp.float32)
         + bl_ref[...]).astype(jnp.bfloat16)
    x = h.reshape(ipb, H, W, h.shape[-1] // (H * W))
    ys = _parity_matmuls(x, w_ref, H, W, ipb)
    _stats4(s_ref, q_ref, ys)
    _store_d2s(o_ref, ys, H, W, ipb)


def _deconv_body(x_ref, sc_ref, sh_ref, w_ref, o_ref, s_ref, q_ref, *,
                 H, W, ipb):
    x = _affine_relu(x_ref[...], sc_ref, sh_ref)
    ys = _parity_matmuls(x, w_ref, H, W, ipb)
    _stats4(s_ref, q_ref, ys)
    _store_d2s(o_ref, ys, H, W, ipb)


def _deconv9_body(x_ref, sc_ref, sh_ref, w_ref, o_ref, s_ref, q_ref, *,
                  H, W, ipb):
    x = _affine_relu(x_ref[...], sc_ref, sh_ref)
    y = _col9_matmul(x, w_ref, H, W, ipb).astype(jnp.bfloat16)
    C = y.shape[-1] // 4
    ys = [y[:, p * C:(p + 1) * C] for p in range(4)]
    _stats4(s_ref, q_ref, ys)
    _store_d2s_packed(o_ref, ys, H, W, ipb)


def _packed_matmul(x, w_ref, R, W, ipb):
    cw = _packed_cols(x, R, W, ipb)
    acc = None
    for dh in range(3):
        m = cw[:, dh:dh + R].reshape(ipb * R * W, cw.shape[-1])
        d = jnp.dot(m, w_ref[dh], preferred_element_type=jnp.float32)
        acc = d if acc is None else acc + d
    return acc


def _conv_packed_body(x_ref, sc_ref, sh_ref, w_ref, o_ref, s_ref, q_ref, *,
                      H, W, ipb):
    x = _affine_relu(x_ref[...], sc_ref, sh_ref)
    y = _packed_matmul(x, w_ref, H, W, ipb).astype(jnp.bfloat16)
    _stats4(s_ref, q_ref, [y])
    o_ref[...] = y.reshape(ipb, H, W, y.shape[-1])


def _conv_tanh_packed_body(x_ref, sc_ref, sh_ref, w_ref, o_ref, *,
                           H, W, ipb):
    x = _affine_relu(x_ref[...], sc_ref, sh_ref)
    o_ref[...] = jnp.tanh(_packed_matmul(x, w_ref, H, W, ipb))


def _pick_ipb(N, HW):
    ipb = max(1, min(N // 2, 1024 // HW))
    while N % ipb:
        ipb -= 1
    return ipb


def _launch_conv(body, x, scale, shift, wmat, Cout, out_kind):
    N, H, W, C = x.shape
    ipb = _pick_ipb(N, H * W)
    G = N // ipb
    kern = functools.partial(body, H=H, W=W, ipb=ipb)
    in_specs = [
        pl.BlockSpec((ipb, H, W, C), lambda i: (i, 0, 0, 0)),
        pl.BlockSpec((1, C), lambda i: (0, 0)),
        pl.BlockSpec((1, C), lambda i: (0, 0)),
        pl.BlockSpec(wmat.shape, lambda i: (0,) * wmat.ndim),
    ]
    if out_kind == "d2s":
        out_shapes = [jax.ShapeDtypeStruct(
            (N, H, 2, 2 * W, Cout // 128, 128), jnp.float32)]
        out_specs = [pl.BlockSpec((ipb, H, 2, 2 * W, Cout // 128, 128),
                                  lambda i: (i, 0, 0, 0, 0, 0))]
        stat_c = 4 * Cout
    elif out_kind == "d2s_packed":
        out_shapes = [jax.ShapeDtypeStruct((N, H, 2, W, 2 * Cout),
                                           jnp.bfloat16)]
        out_specs = [pl.BlockSpec((ipb, H, 2, W, 2 * Cout),
                                  lambda i: (i, 0, 0, 0, 0))]
        stat_c = 4 * Cout
    elif out_kind == "plane":
        out_shapes = [jax.ShapeDtypeStruct((N, H, W, Cout), jnp.bfloat16)]
        out_specs = [pl.BlockSpec((ipb, H, W, Cout),
                                  lambda i: (i, 0, 0, 0))]
        stat_c = Cout
    else:
        out_shapes = [jax.ShapeDtypeStruct((N * H * W, Cout), jnp.float32)]
        out_specs = [pl.BlockSpec((ipb * H * W, Cout), lambda i: (i, 0))]
        stat_c = 0
    if stat_c:
        out_shapes += [jax.ShapeDtypeStruct((G, 1, stat_c), jnp.float32)] * 2
        out_specs += [pl.BlockSpec((1, 1, stat_c), lambda i: (i, 0, 0))] * 2
    out = pl.pallas_call(
        kern,
        out_shape=tuple(out_shapes),
        grid=(G,),
        in_specs=in_specs,
        out_specs=tuple(out_specs),
        compiler_params=pltpu.CompilerParams(
            dimension_semantics=("parallel",)),
    )(x, scale, shift, wmat)
    if out_kind == "d2s":
        return out[0].reshape(N, 2 * H, 2 * W, Cout), out[1], out[2]
    if out_kind == "d2s_packed":
        return out[0].reshape(N, 2 * H, W, 2 * Cout), out[1], out[2]
    return out if stat_c else out[0]


def kernel(emb, lin_w, lin_b, ct1_w, ct2_w, ct3_w, ct4_w, c5_w, c6_w,
           bn1_g, bn1_b, bn2_g, bn2_b, bn3_g, bn3_b, bn4_g, bn4_b,
           bn5_g, bn5_b, z, labels):
    B = z.shape[0]
    ngf = c5_w.shape[0]
    C0 = 8 * ngf
    nc = c6_w.shape[0]

    a = jnp.concatenate([emb[labels], z.reshape(B, -1)],
                        axis=1).astype(jnp.bfloat16)
    wl = lin_w.reshape(C0, 16, -1).transpose(1, 0, 2).reshape(16 * C0, -1).T
    bl = lin_b.reshape(C0, 16).T.reshape(1, 16 * C0)

    H, W = 4, 4
    Cout = ct1_w.shape[1]
    ipb = _pick_ipb(B, H * W)
    G = B // ipb
    out1 = pl.pallas_call(
        functools.partial(_lin_deconv_body, H=H, W=W, ipb=ipb),
        out_shape=(
            jax.ShapeDtypeStruct((B, H, 2, 2 * W, Cout // 128, 128),
                                 jnp.float32),
            jax.ShapeDtypeStruct((G, 1, 4 * Cout), jnp.float32),
            jax.ShapeDtypeStruct((G, 1, 4 * Cout), jnp.float32),
        ),
        grid=(G,),
        in_specs=[
            pl.BlockSpec((ipb, a.shape[1]), lambda i: (i, 0)),
            pl.BlockSpec(wl.shape, lambda i: (0, 0)),
            pl.BlockSpec((1, 16 * C0), lambda i: (0, 0)),
            pl.BlockSpec((4, 2, 2 * C0, Cout), lambda i: (0, 0, 0, 0)),
        ],
        out_specs=(
            pl.BlockSpec((ipb, H, 2, 2 * W, Cout // 128, 128),
                         lambda i: (i, 0, 0, 0, 0, 0)),
            pl.BlockSpec((1, 1, 4 * Cout), lambda i: (i, 0, 0)),
            pl.BlockSpec((1, 1, 4 * Cout), lambda i: (i, 0, 0)),
        ),
        compiler_params=pltpu.CompilerParams(
            dimension_semantics=("parallel",)),
    )(a, wl.astype(jnp.bfloat16), bl.astype(jnp.float32),
      _parity_weights(ct1_w))
    x = out1[0].reshape(B, 2 * H, 2 * W, Cout)
    s, q = out1[1], out1[2]

    def coeffs(s, q, count, gamma, beta, fold4):
        s = s.sum(axis=(0, 1))
        q = q.sum(axis=(0, 1))
        if fold4:
            Cc = gamma.shape[0]
            s = s.reshape(4, Cc).sum(axis=0)
            q = q.reshape(4, Cc).sum(axis=0)
        sc, sh = _bn_scale_shift(s, q, count, gamma, beta)
        return sc.reshape(1, -1), sh.reshape(1, -1)

    for w_t, (g, b) in ((ct2_w, (bn1_g, bn1_b)), (ct3_w, (bn2_g, bn2_b))):
        N, H2, W2, C = x.shape
        sc, sh = coeffs(s, q, float(N * H2 * W2), g, b, True)
        Cout = w_t.shape[1]
        x, s, q = _launch_conv(
            _deconv_body, x, sc, sh, _parity_weights(w_t), Cout, "d2s")

    N, H2, W2, C = x.shape
    sc, sh = coeffs(s, q, float(N * H2 * W2), bn3_g, bn3_b, True)
    Cout = ct4_w.shape[1]
    x, s, q = _launch_conv(
        _deconv9_body, x, sc, sh, _dense_deconv_weights(ct4_w), Cout,
        "d2s_packed")

    N, R, Wp, C2 = x.shape
    cnt = float(N * R * Wp * 2)
    sc, sh = coeffs(s, q, cnt, bn4_g, bn4_b, True)
    sc = jnp.concatenate([sc, sc], axis=1)
    sh = jnp.concatenate([sh, sh], axis=1)
    x, s, q = _launch_conv(
        _conv_packed_body, x, sc, sh, _packed_conv_weights(c5_w),
        2 * ngf, "plane")

    s = s.sum(axis=(0, 1)).reshape(2, ngf).sum(axis=0)
    q_ = q.sum(axis=(0, 1)).reshape(2, ngf).sum(axis=0)
    sc, sh = _bn_scale_shift(s, q_, cnt, bn5_g, bn5_b)
    sc = jnp.concatenate([sc, sc]).reshape(1, -1)
    sh = jnp.concatenate([sh, sh]).reshape(1, -1)
    out = _launch_conv(
        _conv_tanh_packed_body, x, sc, sh, _packed_conv_weights(c6_w),
        2 * nc, "rows")
    img = out.reshape(N, R, 2 * Wp, nc)
    return img.transpose(0, 3, 1, 2)
